# baseline (speedup 1.0000x reference)
import functools

import jax
import jax.numpy as jnp
from jax import lax
from jax.experimental import pallas as pl
from jax.experimental.pallas import tpu as pltpu
from jax.experimental.pallas import tpu_sc as plsc

N = 10000
NPAD = 10240
E = 320000
D = 128
NC = 2
NS = 16
NW = NC * NS
CHUNK = 80
EPAD = 327680
EPW = EPAD // NW
NCHUNK = EPW // CHUNK
NB = 2
NROUND = NCHUNK // NB
RPT = NPAD // NS
DCHUNK = 128
DNCHUNK = EPW // DCHUNK
DFIRE = 16

_mesh = plsc.VectorSubcoreMesh(core_axis_name="c", subcore_axis_name="s")


@functools.partial(
    pl.kernel,
    out_type=jax.ShapeDtypeStruct((NC, NPAD), jnp.float32),
    mesh=_mesh,
    scratch_types=[
        pltpu.VMEM((DNCHUNK, DCHUNK), jnp.int32),
        pltpu.VMEM((DCHUNK,), jnp.float32),
        pltpu.VMEM((RPT,), jnp.float32),
        pltpu.VMEM_SHARED((NPAD,), jnp.float32),
        pltpu.SemaphoreType.DMA,
    ],
)
def _deg_kernel(dst2_hbm, out_hbm, didx, ones_v, zeros_v, acc_s, sem):
    c = lax.axis_index("c")
    s = lax.axis_index("s")
    w = c * NS + s

    for k in range(DCHUNK // 16):
        ones_v[pl.ds(k * 16, 16)] = jnp.ones((16,), jnp.float32)

    def zbody(i, carry):
        zeros_v[pl.ds(i * 16, 16)] = jnp.zeros((16,), jnp.float32)
        return carry

    lax.fori_loop(0, RPT // 16, zbody, 0)
    pltpu.sync_copy(zeros_v, acc_s.at[pl.ds(s * RPT, RPT)])
    pltpu.sync_copy(dst2_hbm.at[w], didx)
    plsc.subcore_barrier()

    def rnd(r, carry):
        def fire(i, carry):
            pltpu.async_copy(ones_v, acc_s.at[didx.at[r * DFIRE + i]], sem, add=True)
            return carry

        lax.fori_loop(0, DFIRE, fire, 0)

        def drain(i, carry):
            pltpu.make_async_copy(ones_v, acc_s.at[didx.at[0]], sem).wait()
            return carry

        lax.fori_loop(0, DFIRE, drain, 0)
        return carry

    lax.fori_loop(0, DNCHUNK // DFIRE, rnd, 0)
    plsc.subcore_barrier()
    pltpu.sync_copy(acc_s.at[pl.ds(s * RPT, RPT)], out_hbm.at[c, pl.ds(s * RPT, RPT)])


@functools.partial(
    pl.kernel,
    out_type=jax.ShapeDtypeStruct((NC, NPAD, D), jnp.float32),
    mesh=_mesh,
    scratch_types=(
        [pltpu.VMEM((4, CHUNK), jnp.int32)] * (2 * NB)
        + [pltpu.VMEM((CHUNK, D), jnp.float32)] * NB
        + [pltpu.SemaphoreType.DMA((4,))] * NB
        + [pltpu.SemaphoreType.DMA] * (2 * NB)
        + [pltpu.VMEM_SHARED((NPAD, D), jnp.float32)]
    ),
)
def _segsum_kernel(src_hbm, dst_hbm, y_hbm, out_hbm, *scratch):
    sidx = scratch[0:NB]
    didx = scratch[NB:2 * NB]
    rows = scratch[2 * NB:3 * NB]
    sem_i = scratch[3 * NB:4 * NB]
    sem_g = scratch[4 * NB:5 * NB]
    sem_s = scratch[5 * NB:6 * NB]
    acc_s = scratch[6 * NB]

    c = lax.axis_index("c")
    s = lax.axis_index("s")
    w = c * NS + s

    def zbody(i, carry):
        for j in range(D // 16):
            rows[0][i, pl.ds(j * 16, 16)] = jnp.zeros((16,), jnp.float32)
        return carry

    lax.fori_loop(0, CHUNK, zbody, 0)
    for k in range(RPT // CHUNK):
        pltpu.sync_copy(rows[0], acc_s.at[pl.ds(s * RPT + k * CHUNK, CHUNK)])
    plsc.subcore_barrier()

    def idx_load(b, q, r):
        off = w * EPW + (r * NB + b) * CHUNK
        pltpu.async_copy(src_hbm.at[pl.ds(off, CHUNK)], sidx[b].at[q], sem_i[b].at[q])
        pltpu.async_copy(dst_hbm.at[pl.ds(off, CHUNK)], didx[b].at[q], sem_i[b].at[q])

    def wait_idx(b, q):
        pltpu.make_async_copy(src_hbm.at[pl.ds(0, CHUNK)], sidx[b].at[q], sem_i[b].at[q]).wait()
        pltpu.make_async_copy(dst_hbm.at[pl.ds(0, CHUNK)], didx[b].at[q], sem_i[b].at[q]).wait()

    def gather(b, q):
        pltpu.async_copy(y_hbm.at[sidx[b].at[q]], rows[b], sem_g[b])

    def wait_gather(b):
        pltpu.make_async_copy(y_hbm.at[sidx[b].at[0]], rows[b], sem_g[b]).wait()

    def scatter(b, q):
        pltpu.async_copy(rows[b], acc_s.at[didx[b].at[q]], sem_s[b], add=True)

    def wait_scatter(b):
        pltpu.make_async_copy(rows[b], acc_s.at[didx[b].at[0]], sem_s[b]).wait()

    for b in range(NB):
        idx_load(b, 0, 0)
        idx_load(b, 1, 1)
    for b in range(NB):
        wait_idx(b, 0)
        gather(b, 0)

    def body(i, carry):
        r0 = i * 4
        for k in range(4):
            for b in range(NB):
                wait_gather(b)
                scatter(b, k)
            for b in range(NB):
                idx_load(b, (k + 2) % 4, r0 + k + 2)
            for b in range(NB):
                wait_scatter(b)
                wait_idx(b, (k + 1) % 4)
                gather(b, (k + 1) % 4)
        return carry

    lax.fori_loop(0, NROUND // 4 - 1, body, 0)
    for k in range(4):
        for b in range(NB):
            wait_gather(b)
            scatter(b, k)
        if k < 2:
            for b in range(NB):
                idx_load(b, k + 2, NROUND - 2 + k)
        if k < 3:
            for b in range(NB):
                wait_scatter(b)
                wait_idx(b, k + 1)
                gather(b, k + 1)
        else:
            for b in range(NB):
                wait_scatter(b)
    plsc.subcore_barrier()
    pltpu.sync_copy(acc_s.at[pl.ds(s * RPT, RPT)], out_hbm.at[c, pl.ds(s * RPT, RPT)])


BN = 1000
_PREC = lax.Precision.HIGHEST


def _dis_of(dp_ref):
    deg = dp_ref[0] + dp_ref[1] + 1.0
    return lax.rsqrt(deg)


def _xw_body(dp_ref, x_ref, w_ref, o_ref):
    dis = _dis_of(dp_ref)
    xw = jnp.dot(x_ref[...], w_ref[...], preferred_element_type=jnp.float32,
                 precision=_PREC)
    o_ref[...] = xw * dis


_xw_kernel = pl.pallas_call(
    _xw_body,
    grid=(N // BN,),
    in_specs=[
        pl.BlockSpec((NC, BN, 1), lambda i: (0, i, 0)),
        pl.BlockSpec((BN, D), lambda i: (i, 0)),
        pl.BlockSpec((D, D), lambda i: (0, 0)),
    ],
    out_specs=pl.BlockSpec((BN, D), lambda i: (i, 0)),
    out_shape=jax.ShapeDtypeStruct((N, D), jnp.float32),
)


def _mid_body(dp_ref, z_ref, y_ref, b_ref, w_ref, o_ref):
    dis = _dis_of(dp_ref)
    t = (z_ref[0] + z_ref[1] + y_ref[...]) * dis + b_ref[...]
    h = jnp.maximum(t, 0.0)
    o_ref[...] = jnp.dot(h, w_ref[...], preferred_element_type=jnp.float32,
                         precision=_PREC) * dis


_mid_kernel = pl.pallas_call(
    _mid_body,
    grid=(N // BN,),
    in_specs=[
        pl.BlockSpec((NC, BN, 1), lambda i: (0, i, 0)),
        pl.BlockSpec((NC, BN, D), lambda i: (0, i, 0)),
        pl.BlockSpec((BN, D), lambda i: (i, 0)),
        pl.BlockSpec((1, D), lambda i: (0, 0)),
        pl.BlockSpec((D, D), lambda i: (0, 0)),
    ],
    out_specs=pl.BlockSpec((BN, D), lambda i: (i, 0)),
    out_shape=jax.ShapeDtypeStruct((N, D), jnp.float32),
)


def _fin_body(dp_ref, z_ref, y_ref, b_ref, o_ref):
    dis = _dis_of(dp_ref)
    o_ref[...] = (z_ref[0] + z_ref[1] + y_ref[...]) * dis + b_ref[...]


_fin_kernel = pl.pallas_call(
    _fin_body,
    grid=(N // BN,),
    in_specs=[
        pl.BlockSpec((NC, BN, 1), lambda i: (0, i, 0)),
        pl.BlockSpec((NC, BN, D), lambda i: (0, i, 0)),
        pl.BlockSpec((BN, D), lambda i: (i, 0)),
        pl.BlockSpec((1, D), lambda i: (0, 0)),
    ],
    out_specs=pl.BlockSpec((BN, D), lambda i: (i, 0)),
    out_shape=jax.ShapeDtypeStruct((N, D), jnp.float32),
)


def kernel(x, edge_index, W1, b1, W2, b2):
    ei = edge_index.astype(jnp.int32)
    npadding = EPAD - E
    pad_dst = N + (jnp.arange(npadding, dtype=jnp.int32) % (NPAD - N))
    src = jnp.concatenate([ei[0], jnp.zeros((npadding,), jnp.int32)])
    dst = jnp.concatenate([ei[1], pad_dst])
    b1r = b1.reshape(1, D)
    b2r = b2.reshape(1, D)

    deg_parts = _deg_kernel(dst.reshape(NW, DNCHUNK, DCHUNK))
    dp = deg_parts[:, :N].reshape(NC, N, 1)

    y1 = _xw_kernel(dp, x, W1)
    z1 = _segsum_kernel(src, dst, y1)[:, :N, :]
    y2 = _mid_kernel(dp, z1, y1, b1r, W2)
    z2 = _segsum_kernel(src, dst, y2)[:, :N, :]
    return _fin_kernel(dp, z2, y2, b2r)


# speedup vs baseline: 1.0421x; 1.0421x over previous
import functools

import jax
import jax.numpy as jnp
from jax import lax
from jax.experimental import pallas as pl
from jax.experimental.pallas import tpu as pltpu
from jax.experimental.pallas import tpu_sc as plsc

N = 10000
NPAD = 10240
E = 320000
D = 128
NC = 2
NS = 16
NW = NC * NS
CHUNK = 80
EPAD = 327680
EPW = EPAD // NW
NCHUNK = EPW // CHUNK
NB = 2
NROUND = NCHUNK // NB
RPT = NPAD // NS
DCHUNK = 128
DNCHUNK = EPW // DCHUNK
DFIRE = 16

_mesh = plsc.VectorSubcoreMesh(core_axis_name="c", subcore_axis_name="s")


@functools.partial(
    pl.kernel,
    out_type=jax.ShapeDtypeStruct((NC, NPAD), jnp.float32),
    mesh=_mesh,
    scratch_types=[
        pltpu.VMEM((DNCHUNK, DCHUNK), jnp.int32),
        pltpu.VMEM((DCHUNK,), jnp.float32),
        pltpu.VMEM((RPT,), jnp.float32),
        pltpu.VMEM_SHARED((NPAD,), jnp.float32),
        pltpu.SemaphoreType.DMA,
    ],
)
def _deg_kernel(dst2_hbm, out_hbm, didx, ones_v, zeros_v, acc_s, sem):
    c = lax.axis_index("c")
    s = lax.axis_index("s")
    w = c * NS + s

    for k in range(DCHUNK // 16):
        ones_v[pl.ds(k * 16, 16)] = jnp.ones((16,), jnp.float32)

    def zbody(i, carry):
        zeros_v[pl.ds(i * 16, 16)] = jnp.zeros((16,), jnp.float32)
        return carry

    lax.fori_loop(0, RPT // 16, zbody, 0)
    pltpu.sync_copy(zeros_v, acc_s.at[pl.ds(s * RPT, RPT)])
    pltpu.sync_copy(dst2_hbm.at[w], didx)
    plsc.subcore_barrier()

    def rnd(r, carry):
        def fire(i, carry):
            pltpu.async_copy(ones_v, acc_s.at[didx.at[r * DFIRE + i]], sem, add=True)
            return carry

        lax.fori_loop(0, DFIRE, fire, 0)

        def drain(i, carry):
            pltpu.make_async_copy(ones_v, acc_s.at[didx.at[0]], sem).wait()
            return carry

        lax.fori_loop(0, DFIRE, drain, 0)
        return carry

    lax.fori_loop(0, DNCHUNK // DFIRE, rnd, 0)
    plsc.subcore_barrier()
    pltpu.sync_copy(acc_s.at[pl.ds(s * RPT, RPT)], out_hbm.at[c, pl.ds(s * RPT, RPT)])


@functools.partial(
    pl.kernel,
    out_type=jax.ShapeDtypeStruct((NC, NPAD, D), jnp.float32),
    mesh=_mesh,
    scratch_types=(
        [pltpu.VMEM((4, CHUNK), jnp.int32)] * (2 * NB)
        + [pltpu.VMEM((CHUNK, D), jnp.float32)] * NB
        + [pltpu.SemaphoreType.DMA((4,))] * NB
        + [pltpu.SemaphoreType.DMA] * NB
        + [pltpu.VMEM_SHARED((NPAD, D), jnp.float32)]
    ),
)
def _segsum_kernel(src_hbm, dst_hbm, y_hbm, out_hbm, *scratch):
    sidx = scratch[0:NB]
    didx = scratch[NB:2 * NB]
    rows = scratch[2 * NB:3 * NB]
    sem_i = scratch[3 * NB:4 * NB]
    sem_g = scratch[4 * NB:5 * NB]
    acc_s = scratch[5 * NB]

    c = lax.axis_index("c")
    s = lax.axis_index("s")
    w = c * NS + s

    def zbody(i, carry):
        for j in range(D // 16):
            rows[0][i, pl.ds(j * 16, 16)] = jnp.zeros((16,), jnp.float32)
        return carry

    lax.fori_loop(0, CHUNK, zbody, 0)
    for k in range(RPT // CHUNK):
        pltpu.sync_copy(rows[0], acc_s.at[pl.ds(s * RPT + k * CHUNK, CHUNK)])
    plsc.subcore_barrier()

    def idx_load(b, q, r):
        off = w * EPW + (r * NB + b) * CHUNK
        pltpu.async_copy(src_hbm.at[pl.ds(off, CHUNK)], sidx[b].at[q], sem_i[b].at[q])
        pltpu.async_copy(dst_hbm.at[pl.ds(off, CHUNK)], didx[b].at[q], sem_i[b].at[q])

    def wait_idx(b, q):
        pltpu.make_async_copy(src_hbm.at[pl.ds(0, CHUNK)], sidx[b].at[q], sem_i[b].at[q]).wait()
        pltpu.make_async_copy(dst_hbm.at[pl.ds(0, CHUNK)], didx[b].at[q], sem_i[b].at[q]).wait()

    def gather(b, q):
        pltpu.async_copy(y_hbm.at[sidx[b].at[q]], rows[b], sem_g[b])

    def wait_gather(b):
        pltpu.make_async_copy(y_hbm.at[sidx[b].at[0]], rows[b], sem_g[b]).wait()

    def scatter(b, q):
        pltpu.sync_copy(rows[b], acc_s.at[didx[b].at[q]], add=True)

    for b in range(NB):
        idx_load(b, 0, 0)
        idx_load(b, 1, 1)
    for b in range(NB):
        wait_idx(b, 0)
        gather(b, 0)

    def body(i, carry):
        r0 = i * 4
        for k in range(4):
            for b in range(NB):
                wait_gather(b)
                scatter(b, k)
                wait_idx(b, (k + 1) % 4)
                gather(b, (k + 1) % 4)
            for b in range(NB):
                idx_load(b, (k + 2) % 4, r0 + k + 2)
        return carry

    lax.fori_loop(0, NROUND // 4 - 1, body, 0)
    for k in range(4):
        for b in range(NB):
            wait_gather(b)
            scatter(b, k)
            if k < 3:
                wait_idx(b, k + 1)
                gather(b, k + 1)
        if k < 2:
            for b in range(NB):
                idx_load(b, k + 2, NROUND - 2 + k)
    plsc.subcore_barrier()
    pltpu.sync_copy(acc_s.at[pl.ds(s * RPT, RPT)], out_hbm.at[c, pl.ds(s * RPT, RPT)])


BN = 1000
_PREC = lax.Precision.HIGHEST


def _dis_of(dp_ref):
    deg = dp_ref[0] + dp_ref[1] + 1.0
    return lax.rsqrt(deg)


def _xw_body(dp_ref, x_ref, w_ref, o_ref):
    dis = _dis_of(dp_ref)
    xw = jnp.dot(x_ref[...], w_ref[...], preferred_element_type=jnp.float32,
                 precision=_PREC)
    o_ref[...] = xw * dis


_xw_kernel = pl.pallas_call(
    _xw_body,
    grid=(N // BN,),
    in_specs=[
        pl.BlockSpec((NC, BN, 1), lambda i: (0, i, 0)),
        pl.BlockSpec((BN, D), lambda i: (i, 0)),
        pl.BlockSpec((D, D), lambda i: (0, 0)),
    ],
    out_specs=pl.BlockSpec((BN, D), lambda i: (i, 0)),
    out_shape=jax.ShapeDtypeStruct((N, D), jnp.float32),
)


def _mid_body(dp_ref, z_ref, y_ref, b_ref, w_ref, o_ref):
    dis = _dis_of(dp_ref)
    t = (z_ref[0] + z_ref[1] + y_ref[...]) * dis + b_ref[...]
    h = jnp.maximum(t, 0.0)
    o_ref[...] = jnp.dot(h, w_ref[...], preferred_element_type=jnp.float32,
                         precision=_PREC) * dis


_mid_kernel = pl.pallas_call(
    _mid_body,
    grid=(N // BN,),
    in_specs=[
        pl.BlockSpec((NC, BN, 1), lambda i: (0, i, 0)),
        pl.BlockSpec((NC, BN, D), lambda i: (0, i, 0)),
        pl.BlockSpec((BN, D), lambda i: (i, 0)),
        pl.BlockSpec((1, D), lambda i: (0, 0)),
        pl.BlockSpec((D, D), lambda i: (0, 0)),
    ],
    out_specs=pl.BlockSpec((BN, D), lambda i: (i, 0)),
    out_shape=jax.ShapeDtypeStruct((N, D), jnp.float32),
)


def _fin_body(dp_ref, z_ref, y_ref, b_ref, o_ref):
    dis = _dis_of(dp_ref)
    o_ref[...] = (z_ref[0] + z_ref[1] + y_ref[...]) * dis + b_ref[...]


_fin_kernel = pl.pallas_call(
    _fin_body,
    grid=(N // BN,),
    in_specs=[
        pl.BlockSpec((NC, BN, 1), lambda i: (0, i, 0)),
        pl.BlockSpec((NC, BN, D), lambda i: (0, i, 0)),
        pl.BlockSpec((BN, D), lambda i: (i, 0)),
        pl.BlockSpec((1, D), lambda i: (0, 0)),
    ],
    out_specs=pl.BlockSpec((BN, D), lambda i: (i, 0)),
    out_shape=jax.ShapeDtypeStruct((N, D), jnp.float32),
)


def kernel(x, edge_index, W1, b1, W2, b2):
    ei = edge_index.astype(jnp.int32)
    npadding = EPAD - E
    pad_dst = N + (jnp.arange(npadding, dtype=jnp.int32) % (NPAD - N))
    src = jnp.concatenate([ei[0], jnp.zeros((npadding,), jnp.int32)])
    dst = jnp.concatenate([ei[1], pad_dst])
    b1r = b1.reshape(1, D)
    b2r = b2.reshape(1, D)

    deg_parts = _deg_kernel(dst.reshape(NW, DNCHUNK, DCHUNK))
    dp = deg_parts[:, :N].reshape(NC, N, 1)

    y1 = _xw_kernel(dp, x, W1)
    z1 = _segsum_kernel(src, dst, y1)[:, :N, :]
    y2 = _mid_kernel(dp, z1, y1, b1r, W2)
    z2 = _segsum_kernel(src, dst, y2)[:, :N, :]
    return _fin_kernel(dp, z2, y2, b2r)
